# Initial kernel scaffold; baseline (speedup 1.0000x reference)
import jax
import jax.numpy as jnp
from jax.experimental import pallas as pl


def kernel(tokens, table):
    raise NotImplementedError("write your pallas kernel here")


# speedup vs baseline: 7.6511x; 7.6511x over previous
import functools

import jax
import jax.numpy as jnp
from jax import lax
from jax.experimental import pallas as pl
from jax.experimental.pallas import tpu as pltpu
from jax.experimental.pallas import tpu_sc as plsc

B, S, D = 4096, 200, 128
NC, NS, L = 2, 16, 16
NW = NC * NS
BPW = B // NW
HALF = 104
NCHUNK = D // L

_mesh = plsc.VectorSubcoreMesh(core_axis_name="c", subcore_axis_name="s")


@functools.partial(
    pl.kernel,
    mesh=_mesh,
    out_type=jax.ShapeDtypeStruct((B, D), jnp.float32),
    scratch_types=[
        pltpu.VMEM((BPW * S,), jnp.int32),
        pltpu.VMEM((S, D), jnp.float32),
        pltpu.VMEM((BPW, D), jnp.float32),
        pltpu.SemaphoreType.DMA,
    ],
)
def _embed_mean(tokens_hbm, table_hbm, out_hbm, tok_v, rows_v, out_v, sem):
    wid = lax.axis_index("s") * NC + lax.axis_index("c")
    base = wid * BPW
    pltpu.sync_copy(tokens_hbm.at[pl.ds(base * S, BPW * S)], tok_v)

    def row_body(i, carry):
        c0 = pltpu.async_copy(table_hbm.at[tok_v.at[pl.ds(i * S, HALF)]],
                              rows_v.at[pl.ds(0, HALF)], sem)
        c1 = pltpu.async_copy(table_hbm.at[tok_v.at[pl.ds(i * S + HALF, S - HALF)]],
                              rows_v.at[pl.ds(HALF, S - HALF)], sem)
        c0.wait()
        c1.wait()

        def acc_body(r, accs):
            return tuple(accs[k] + rows_v[r, pl.ds(k * L, L)]
                         for k in range(NCHUNK))

        accs = lax.fori_loop(
            0, S, acc_body,
            tuple(jnp.zeros((L,), jnp.float32) for _ in range(NCHUNK)))
        for k in range(NCHUNK):
            out_v[i, pl.ds(k * L, L)] = accs[k] * (1.0 / S)
        return carry

    lax.fori_loop(0, BPW, row_body, 0)
    pltpu.sync_copy(out_v, out_hbm.at[pl.ds(base, BPW)])


def kernel(tokens, table):
    tok_flat = tokens.reshape(-1).astype(jnp.int32)
    return _embed_mean(tok_flat, table)


# speedup vs baseline: 13.4827x; 1.7622x over previous
import functools

import jax
import jax.numpy as jnp
from jax import lax
from jax.experimental import pallas as pl
from jax.experimental.pallas import tpu as pltpu
from jax.experimental.pallas import tpu_sc as plsc

B, S, D = 4096, 200, 128
NC, NS, L = 2, 16, 16
NW = NC * NS
BPW = B // NW
HALF = 104
NCHUNK = D // L
UNROLL = 4

_mesh = plsc.VectorSubcoreMesh(core_axis_name="c", subcore_axis_name="s")


def _fire(table_hbm, tok_v, rbuf, sem, i):
    pltpu.async_copy(table_hbm.at[tok_v.at[pl.ds(i * S, HALF)]],
                     rbuf.at[pl.ds(0, HALF)], sem)
    pltpu.async_copy(table_hbm.at[tok_v.at[pl.ds(i * S + HALF, S - HALF)]],
                     rbuf.at[pl.ds(HALF, S - HALF)], sem)


def _wait(table_hbm, tok_v, rbuf, sem, i):
    pltpu.make_async_copy(table_hbm.at[tok_v.at[pl.ds(i * S, HALF)]],
                          rbuf.at[pl.ds(0, HALF)], sem).wait()
    pltpu.make_async_copy(table_hbm.at[tok_v.at[pl.ds(i * S + HALF, S - HALF)]],
                          rbuf.at[pl.ds(HALF, S - HALF)], sem).wait()


def _accumulate(rbuf, out_v, i):
    def acc_body(r, accs):
        new = []
        for k in range(NCHUNK):
            s = accs[k]
            for u in range(UNROLL):
                s = s + rbuf[r * UNROLL + u, pl.ds(k * L, L)]
            new.append(s)
        return tuple(new)

    accs = lax.fori_loop(
        0, S // UNROLL, acc_body,
        tuple(jnp.zeros((L,), jnp.float32) for _ in range(NCHUNK)))
    for k in range(NCHUNK):
        out_v[i, pl.ds(k * L, L)] = accs[k] * (1.0 / S)


@functools.partial(
    pl.kernel,
    mesh=_mesh,
    out_type=jax.ShapeDtypeStruct((B, D), jnp.float32),
    scratch_types=[
        pltpu.VMEM((BPW * S,), jnp.int32),
        pltpu.VMEM((S, D), jnp.float32),
        pltpu.VMEM((S, D), jnp.float32),
        pltpu.VMEM((BPW, D), jnp.float32),
        pltpu.SemaphoreType.DMA,
        pltpu.SemaphoreType.DMA,
    ],
)
def _embed_mean(tokens_hbm, table_hbm, out_hbm,
                tok_v, buf0, buf1, out_v, sem0, sem1):
    wid = lax.axis_index("s") * NC + lax.axis_index("c")
    base = wid * BPW
    pltpu.sync_copy(tokens_hbm.at[pl.ds(base * S, BPW * S)], tok_v)

    _fire(table_hbm, tok_v, buf0, sem0, 0)
    _fire(table_hbm, tok_v, buf1, sem1, 1)

    def pair_body(j, carry):
        i2 = j * 2
        _wait(table_hbm, tok_v, buf0, sem0, i2)
        _accumulate(buf0, out_v, i2)

        @pl.when(i2 + 2 < BPW)
        def _():
            _fire(table_hbm, tok_v, buf0, sem0, i2 + 2)

        _wait(table_hbm, tok_v, buf1, sem1, i2 + 1)
        _accumulate(buf1, out_v, i2 + 1)

        @pl.when(i2 + 3 < BPW)
        def _():
            _fire(table_hbm, tok_v, buf1, sem1, i2 + 3)

        return carry

    lax.fori_loop(0, BPW // 2, pair_body, 0)
    pltpu.sync_copy(out_v, out_hbm.at[pl.ds(base, BPW)])


def kernel(tokens, table):
    tok_flat = tokens.reshape(-1).astype(jnp.int32)
    return _embed_mean(tok_flat, table)
